# Initial kernel scaffold; baseline (speedup 1.0000x reference)
import jax
import jax.numpy as jnp
from jax.experimental import pallas as pl


def kernel(x, edge_index, edge_weight, W1, b1, W2, b2, W3, b3, bn1_gamma, bn1_beta, bn1_mean, bn1_var, bn2_gamma, bn2_beta, bn2_mean, bn2_var):
    raise NotImplementedError("write your pallas kernel here")


# speedup vs baseline: 8.8236x; 8.8236x over previous
import functools

import jax
import jax.numpy as jnp
from jax import lax
from jax.experimental import pallas as pl
from jax.experimental.pallas import tpu as pltpu
from jax.experimental.pallas import tpu_sc as plsc

N = 10000
NP = 10240
E = 320000
D = 128
C = 16
EPS = 1e-5

NC = 2
NS = 16
NW = NC * NS
EPW = E // NW
CH = 80
NCHUNK = EPW // CH
RPS = NP // NS

RB = 1024
GRID = NP // RB

@functools.lru_cache(maxsize=None)
def _mesh():
    return plsc.VectorSubcoreMesh(
        core_axis_name="c", subcore_axis_name="s",
        num_cores=NC, num_subcores=NS)


def _deg_body(dst_hbm, ew_hbm, deg_out, dstv, ewv, zv, spdeg):
    c = lax.axis_index("c")
    s = lax.axis_index("s")
    wid = s * NC + c

    def zero(i, _):
        zv[pl.ds(i * 16, 16)] = jnp.zeros((16,), jnp.float32)
        return 0
    lax.fori_loop(0, RPS // 16, zero, 0)
    pltpu.sync_copy(zv, spdeg.at[pl.ds(s * RPS, RPS)])
    plsc.subcore_barrier()

    base = wid * EPW

    def chunk(i, _):
        off = pl.multiple_of(base + i * CH, 8)
        pltpu.sync_copy(dst_hbm.at[pl.ds(off, CH)], dstv)
        pltpu.sync_copy(ew_hbm.at[pl.ds(off, CH)], ewv)
        pltpu.sync_copy(ewv, spdeg.at[dstv], add=True)
        return 0
    lax.fori_loop(0, NCHUNK, chunk, 0)
    plsc.subcore_barrier()

    pltpu.sync_copy(spdeg.at[pl.ds(s * RPS, RPS)],
                    deg_out.at[pl.ds(c * NP + s * RPS, RPS)])


@functools.lru_cache(maxsize=None)
def _deg_call():
    return pl.kernel(
        _deg_body,
        out_type=jax.ShapeDtypeStruct((NC * NP,), jnp.float32),
        mesh=_mesh(),
        scratch_types=[
            pltpu.VMEM((CH,), jnp.int32),
            pltpu.VMEM((CH,), jnp.float32),
            pltpu.VMEM((RPS,), jnp.float32),
            pltpu.VMEM_SHARED((NP,), jnp.float32),
        ],
    )


def _msg_body(width, src_hbm, dst_hbm, ew_hbm, h_hbm, s_out,
              sidx, didx, ewv, rows, spacc, sem):
    c = lax.axis_index("c")
    s = lax.axis_index("s")
    wid = s * NC + c
    nz = width // 16

    def zero(i, _):
        r = i // nz
        j = i % nz
        rows[r, pl.ds((i % nz) * 16, 16)] = jnp.zeros((16,), jnp.float32)
        return 0
    lax.fori_loop(0, CH * nz, zero, 0)
    for k in range(RPS // CH):
        pltpu.sync_copy(rows, spacc.at[pl.ds(s * RPS + k * CH, CH)])
    plsc.subcore_barrier()

    base = wid * EPW

    def chunk(i, _):
        off = pl.multiple_of(base + i * CH, 8)
        pltpu.sync_copy(src_hbm.at[pl.ds(off, CH)], sidx)
        pltpu.sync_copy(dst_hbm.at[pl.ds(off, CH)], didx)
        pltpu.sync_copy(ew_hbm.at[pl.ds(off, CH)], ewv)
        pltpu.async_copy(h_hbm.at[sidx], rows, sem).wait()

        def scale(g, _):
            ewvec = ewv[pl.ds(g * 16, 16)]
            for t in range(16):
                w = ewvec[t]
                r = g * 16 + t
                for j in range(nz):
                    sl = pl.ds(j * 16, 16)
                    rows[r, sl] = rows[r, sl] * w
            return 0
        lax.fori_loop(0, CH // 16, scale, 0)
        pltpu.sync_copy(rows, spacc.at[didx], add=True)
        return 0
    lax.fori_loop(0, NCHUNK, chunk, 0)
    plsc.subcore_barrier()

    pltpu.sync_copy(spacc.at[pl.ds(s * RPS, RPS)],
                    s_out.at[pl.ds(c * NP + s * RPS, RPS)])


@functools.lru_cache(maxsize=None)
def _msg_call(width):
    return pl.kernel(
        functools.partial(_msg_body, width),
        out_type=jax.ShapeDtypeStruct((NC * NP, width), jnp.float32),
        mesh=_mesh(),
        scratch_types=[
            pltpu.VMEM((CH,), jnp.int32),
            pltpu.VMEM((CH,), jnp.int32),
            pltpu.VMEM((CH,), jnp.float32),
            pltpu.VMEM((CH, width), jnp.float32),
            pltpu.VMEM_SHARED((NP, width), jnp.float32),
            pltpu.SemaphoreType.DMA,
        ],
        compiler_params=pltpu.CompilerParams(use_tc_tiling_on_sc=False),
    )


def _dinv(degcol):
    deg = degcol + 1.0
    return jnp.where(deg > 0, lax.rsqrt(jnp.where(deg > 0, deg, 1.0)), 0.0)


def _proj_body(degcol_ref, x_ref, w_ref, hp_ref):
    dinv = _dinv(degcol_ref[...])
    h = jnp.dot(x_ref[...], w_ref[...], preferred_element_type=jnp.float32)
    hp_ref[...] = dinv * h


def _comb_body(s_ref, hp_ref, res_ref, degcol_ref, w_ref, b_ref,
               g_ref, be_ref, m_ref, v_ref, hnext_ref, hpnext_ref):
    dinv = _dinv(degcol_ref[...])
    conv = dinv * (s_ref[0] + s_ref[1] + hp_ref[...]) + b_ref[...]
    scale = g_ref[...] * lax.rsqrt(v_ref[...] + EPS)
    bn = (conv - m_ref[...]) * scale + be_ref[...]
    hnext = jnp.maximum(bn, 0.0) + res_ref[...]
    hnext_ref[...] = hnext
    hpnext_ref[...] = dinv * jnp.dot(hnext, w_ref[...],
                                     preferred_element_type=jnp.float32)


def _final_body(s_ref, tp_ref, degcol_ref, b_ref, out_ref):
    dinv = _dinv(degcol_ref[...])
    out_ref[...] = dinv * (s_ref[0] + s_ref[1] + tp_ref[...]) + b_ref[...]


def _row_spec(w):
    return pl.BlockSpec((RB, w), lambda r: (r, 0))


def _full_spec(shape):
    nd = len(shape)
    return pl.BlockSpec(shape, lambda r: (0,) * nd)


def _part_spec(w):
    return pl.BlockSpec((NC, RB, w), lambda r: (0, r, 0))


_proj_call = pl.pallas_call(
    _proj_body,
    grid=(GRID,),
    in_specs=[_row_spec(1), _row_spec(D), _full_spec((D, D))],
    out_specs=_row_spec(D),
    out_shape=jax.ShapeDtypeStruct((NP, D), jnp.float32),
)


def _make_comb_call(wout):
    return pl.pallas_call(
        functools.partial(_comb_body),
        grid=(GRID,),
        in_specs=[_part_spec(D), _row_spec(D), _row_spec(D), _row_spec(1),
                  _full_spec((D, wout)), _full_spec((1, D)),
                  _full_spec((1, D)), _full_spec((1, D)),
                  _full_spec((1, D)), _full_spec((1, D))],
        out_specs=[_row_spec(D), _row_spec(wout)],
        out_shape=[jax.ShapeDtypeStruct((NP, D), jnp.float32),
                   jax.ShapeDtypeStruct((NP, wout), jnp.float32)],
    )


_comb_call_128 = _make_comb_call(D)
_comb_call_16 = _make_comb_call(C)

_final_call = pl.pallas_call(
    _final_body,
    grid=(GRID,),
    in_specs=[_part_spec(C), _row_spec(C), _row_spec(1), _full_spec((1, C))],
    out_specs=_row_spec(C),
    out_shape=jax.ShapeDtypeStruct((NP, C), jnp.float32),
)


def kernel(x, edge_index, edge_weight, W1, b1, W2, b2, W3, b3,
           bn1_gamma, bn1_beta, bn1_mean, bn1_var,
           bn2_gamma, bn2_beta, bn2_mean, bn2_var):
    src = edge_index[0]
    dst = edge_index[1]
    x_pad = jnp.pad(x, ((0, NP - N), (0, 0)))

    deg_flat = _deg_call()(dst, edge_weight)
    degcol = (deg_flat[:NP] + deg_flat[NP:]).reshape(NP, 1)

    b1r = b1.reshape(1, D)
    b2r = b2.reshape(1, D)
    b3r = b3.reshape(1, C)

    hp1 = _proj_call(degcol, x_pad, W1)
    s1 = _msg_call(D)(src, dst, edge_weight, hp1).reshape(NC, NP, D)
    h2, hp2 = _comb_call_128(s1, hp1, x_pad, degcol, W2, b1r,
                             bn1_gamma.reshape(1, D), bn1_beta.reshape(1, D),
                             bn1_mean.reshape(1, D), bn1_var.reshape(1, D))
    s2 = _msg_call(D)(src, dst, edge_weight, hp2).reshape(NC, NP, D)
    _, tp = _comb_call_16(s2, hp2, h2, degcol, W3, b2r,
                          bn2_gamma.reshape(1, D), bn2_beta.reshape(1, D),
                          bn2_mean.reshape(1, D), bn2_var.reshape(1, D))
    s3 = _msg_call(C)(src, dst, edge_weight, tp).reshape(NC, NP, C)
    out = _final_call(s3, tp, degcol, b3r)
    return out[:N]


# speedup vs baseline: 20.6358x; 2.3387x over previous
import functools

import jax
import jax.numpy as jnp
from jax import lax
from jax.experimental import pallas as pl
from jax.experimental.pallas import tpu as pltpu
from jax.experimental.pallas import tpu_sc as plsc

N = 10000
NP = 10240
E = 320000
D = 128
C = 16
EPS = 1e-5

NC = 2
NS = 16
NW = NC * NS
EPW = E // NW
CH = 80
NCHUNK = EPW // CH
SPA = NP
RPA = SPA // NS

RB = 1024
GRID = NP // RB

@functools.lru_cache(maxsize=None)
def _mesh():
    return plsc.VectorSubcoreMesh(
        core_axis_name="c", subcore_axis_name="s",
        num_cores=NC, num_subcores=NS)


def _deg_body(dst_hbm, ew_hbm, deg_out, dstv, ewv, degtile):
    c = lax.axis_index("c")
    s = lax.axis_index("s")
    wid = s * NC + c

    pltpu.sync_copy(dst_hbm.at[pl.ds(wid * EPW, EPW)], dstv)
    pltpu.sync_copy(ew_hbm.at[pl.ds(wid * EPW, EPW)], ewv)

    def zero(i, _):
        degtile[pl.ds(i * 16, 16)] = jnp.zeros((16,), jnp.float32)
        return 0
    lax.fori_loop(0, NP // 16, zero, 0)

    def grp(g, _):
        dv = dstv[pl.ds(g * 16, 16)]
        wv = ewv[pl.ds(g * 16, 16)]
        plsc.addupdate_scatter(degtile, [dv], wv)
        return 0
    lax.fori_loop(0, EPW // 16, grp, 0)

    pltpu.sync_copy(degtile, deg_out.at[pl.ds(wid * NP, NP)])


@functools.lru_cache(maxsize=None)
def _deg_call():
    return pl.kernel(
        _deg_body,
        out_type=jax.ShapeDtypeStruct((NW * NP,), jnp.float32),
        mesh=_mesh(),
        scratch_types=[
            pltpu.VMEM((EPW,), jnp.int32),
            pltpu.VMEM((EPW,), jnp.float32),
            pltpu.VMEM((NP,), jnp.float32),
        ],
        compiler_params=pltpu.CompilerParams(needs_layout_passes=False),
    )


def _msg_body(width, P, src_hbm, dst_hbm, ew_hbm, h_hbm, s_out, *scr):
    sidx = scr[0:P]
    didx = scr[P:2 * P]
    eww = scr[2 * P:3 * P]
    gbuf = scr[3 * P:4 * P]
    sbuf = scr[4 * P:5 * P]
    spacc = scr[5 * P]
    gsem = scr[5 * P + 1:6 * P + 1]
    ssem = scr[6 * P + 1:7 * P + 1]
    isems = scr[7 * P + 1:8 * P + 1]
    isemd = scr[8 * P + 1:9 * P + 1]
    esem = scr[9 * P + 1:10 * P + 1]

    c = lax.axis_index("c")
    s = lax.axis_index("s")
    wid = s * NC + c
    nz = width // 16
    base = wid * EPW

    def ioff(i):
        return pl.multiple_of(base + i * CH, 8)

    def zero(i, _):
        gbuf[0][i // nz, pl.ds((i % nz) * 16, 16)] = jnp.zeros((16,),
                                                              jnp.float32)
        return 0
    lax.fori_loop(0, CH * nz, zero, 0)
    for k in range(RPA // CH):
        pltpu.sync_copy(gbuf[0], spacc.at[pl.ds(s * RPA + k * CH, CH)])
    plsc.subcore_barrier()

    def gissue(idx, buf, sem):
        pltpu.async_copy(h_hbm.at[idx], buf, sem)

    def gwait(idx, buf, sem):
        pltpu.make_async_copy(h_hbm.at[idx], buf, sem).wait()

    def sissue(idx, buf, sem):
        pltpu.async_copy(buf, spacc.at[idx], sem, add=True)

    def swait(idx, buf, sem):
        pltpu.make_async_copy(buf, spacc.at[idx], sem).wait()

    def scale(src_buf, dst_buf, ew_buf):
        def grp(g, _):
            ewvec = ew_buf[pl.ds(g * 16, 16)]
            for t in range(16):
                w = ewvec[t]
                r = g * 16 + t
                for j in range(nz):
                    sl = pl.ds(j * 16, 16)
                    dst_buf[r, sl] = src_buf[r, sl] * w
            return 0
        lax.fori_loop(0, CH // 16, grp, 0)

    last = NCHUNK - 1
    for p in range(P):
        pltpu.sync_copy(src_hbm.at[pl.ds(ioff(p), CH)], sidx[p])
        gissue(sidx[p], gbuf[p], gsem[p])

    def step(i, p):
        nxt = jnp.minimum(i + P, last)
        pltpu.async_copy(ew_hbm.at[pl.ds(ioff(i), CH)], eww[p], esem[p])
        gwait(sidx[p], gbuf[p], gsem[p])
        pltpu.async_copy(src_hbm.at[pl.ds(ioff(nxt), CH)], sidx[p], isems[p])

        @pl.when(i >= P)
        def _():
            swait(didx[p], sbuf[p], ssem[p])
        pltpu.async_copy(dst_hbm.at[pl.ds(ioff(i), CH)], didx[p], isemd[p])
        pltpu.make_async_copy(ew_hbm.at[pl.ds(ioff(i), CH)], eww[p],
                              esem[p]).wait()
        scale(gbuf[p], sbuf[p], eww[p])
        pltpu.make_async_copy(dst_hbm.at[pl.ds(ioff(i), CH)], didx[p],
                              isemd[p]).wait()
        sissue(didx[p], sbuf[p], ssem[p])
        pltpu.make_async_copy(src_hbm.at[pl.ds(ioff(nxt), CH)], sidx[p],
                              isems[p]).wait()
        gissue(sidx[p], gbuf[p], gsem[p])

    def body(k, _):
        for p in range(P):
            step(P * k + p, p)
        return 0
    nmain = NCHUNK // P
    lax.fori_loop(0, nmain, body, 0)

    for q in range(NCHUNK - nmain * P):
        step(nmain * P + q, q)
    for p in range(P):
        swait(didx[p], sbuf[p], ssem[p])
        gwait(sidx[p], gbuf[p], gsem[p])
    plsc.subcore_barrier()

    pltpu.sync_copy(spacc.at[pl.ds(s * RPA, RPA)],
                    s_out.at[pl.ds(c * NP + s * RPA, RPA)])


@functools.lru_cache(maxsize=None)
def _msg_call(width):
    P = 2 if width == D else 4
    scratch = (
        [pltpu.VMEM((CH,), jnp.int32) for _ in range(2 * P)] +
        [pltpu.VMEM((CH,), jnp.float32) for _ in range(P)] +
        [pltpu.VMEM((CH, width), jnp.float32) for _ in range(2 * P)] +
        [pltpu.VMEM_SHARED((SPA, width), jnp.float32)] +
        [pltpu.SemaphoreType.DMA] * (5 * P)
    )
    return pl.kernel(
        functools.partial(_msg_body, width, P),
        out_type=jax.ShapeDtypeStruct((NC * NP, width), jnp.float32),
        mesh=_mesh(),
        scratch_types=scratch,
        compiler_params=pltpu.CompilerParams(use_tc_tiling_on_sc=False,
                                             needs_layout_passes=False),
    )


def _dinv(degcol):
    deg = degcol + 1.0
    return jnp.where(deg > 0, lax.rsqrt(jnp.where(deg > 0, deg, 1.0)), 0.0)


def _proj_body(degcol_ref, x_ref, w_ref, hp_ref):
    dinv = _dinv(degcol_ref[...])
    h = jnp.dot(x_ref[...], w_ref[...], preferred_element_type=jnp.float32)
    hp_ref[...] = dinv * h


def _comb_body(emit_res, s_ref, hp_ref, res_ref, degcol_ref, w_ref, b_ref,
               g_ref, be_ref, m_ref, v_ref, *out_refs):
    dinv = _dinv(degcol_ref[...])
    conv = dinv * (s_ref[0] + s_ref[1] + hp_ref[...]) + b_ref[...]
    scale = g_ref[...] * lax.rsqrt(v_ref[...] + EPS)
    bn = (conv - m_ref[...]) * scale + be_ref[...]
    hnext = jnp.maximum(bn, 0.0) + res_ref[...]
    if emit_res:
        out_refs[0][...] = hnext
    out_refs[-1][...] = dinv * jnp.dot(hnext, w_ref[...],
                                       preferred_element_type=jnp.float32)


def _final_body(s_ref, tp_ref, degcol_ref, b_ref, out_ref):
    dinv = _dinv(degcol_ref[...])
    out_ref[...] = dinv * (s_ref[0] + s_ref[1] + tp_ref[...]) + b_ref[...]


def _row_spec(w):
    return pl.BlockSpec((RB, w), lambda r: (r, 0))


def _full_spec(shape):
    nd = len(shape)
    return pl.BlockSpec(shape, lambda r: (0,) * nd)


def _part_spec(w):
    return pl.BlockSpec((NC, RB, w), lambda r: (0, r, 0))


_proj_call = pl.pallas_call(
    _proj_body,
    grid=(GRID,),
    in_specs=[_row_spec(1), _row_spec(D), _full_spec((D, D))],
    out_specs=_row_spec(D),
    out_shape=jax.ShapeDtypeStruct((NP, D), jnp.float32),
)


def _make_comb_call(wout, emit_res):
    out_specs = [_row_spec(D)] * emit_res + [_row_spec(wout)]
    out_shape = ([jax.ShapeDtypeStruct((NP, D), jnp.float32)] * emit_res +
                 [jax.ShapeDtypeStruct((NP, wout), jnp.float32)])
    return pl.pallas_call(
        functools.partial(_comb_body, emit_res),
        grid=(GRID,),
        in_specs=[_part_spec(D), _row_spec(D), _row_spec(D), _row_spec(1),
                  _full_spec((D, wout)), _full_spec((1, D)),
                  _full_spec((1, D)), _full_spec((1, D)),
                  _full_spec((1, D)), _full_spec((1, D))],
        out_specs=out_specs,
        out_shape=out_shape,
    )


_comb_call_128 = _make_comb_call(D, True)
_comb_call_16 = _make_comb_call(C, False)

_final_call = pl.pallas_call(
    _final_body,
    grid=(GRID,),
    in_specs=[_part_spec(C), _row_spec(C), _row_spec(1), _full_spec((1, C))],
    out_specs=_row_spec(C),
    out_shape=jax.ShapeDtypeStruct((NP, C), jnp.float32),
)


def kernel(x, edge_index, edge_weight, W1, b1, W2, b2, W3, b3,
           bn1_gamma, bn1_beta, bn1_mean, bn1_var,
           bn2_gamma, bn2_beta, bn2_mean, bn2_var):
    src = edge_index[0]
    dst = edge_index[1]
    x_pad = jnp.pad(x, ((0, NP - N), (0, 0)))

    deg_flat = _deg_call()(dst, edge_weight)
    degcol = deg_flat.reshape(NW, NP).sum(axis=0).reshape(NP, 1)

    b1r = b1.reshape(1, D)
    b2r = b2.reshape(1, D)
    b3r = b3.reshape(1, C)

    hp1 = _proj_call(degcol, x_pad, W1)
    s1 = _msg_call(D)(src, dst, edge_weight, hp1).reshape(NC, NP, D)
    h2, hp2 = _comb_call_128(s1, hp1, x_pad, degcol, W2, b1r,
                             bn1_gamma.reshape(1, D), bn1_beta.reshape(1, D),
                             bn1_mean.reshape(1, D), bn1_var.reshape(1, D))
    s2 = _msg_call(D)(src, dst, edge_weight, hp2).reshape(NC, NP, D)
    (tp,) = _comb_call_16(s2, hp2, h2, degcol, W3, b2r,
                          bn2_gamma.reshape(1, D), bn2_beta.reshape(1, D),
                          bn2_mean.reshape(1, D), bn2_var.reshape(1, D))
    s3 = _msg_call(C)(src, dst, edge_weight, tp).reshape(NC, NP, C)
    out = _final_call(s3, tp, degcol, b3r)
    return out[:N]
